# baseline (speedup 1.0000x reference)
import functools

import jax
import jax.numpy as jnp
from jax import lax
from jax.experimental import pallas as pl
from jax.experimental.pallas import tpu as pltpu
from jax.experimental.pallas import tpu_sc as plsc

HDIM = 64
NC = 2
NS = 16
NW = NC * NS
CHUNK = 128


@functools.partial(jax.jit, static_argnames=("n_chunks",))
def _gather(x3, weight, n_chunks):
    mesh = plsc.VectorSubcoreMesh(
        core_axis_name="c", subcore_axis_name="s",
        num_cores=NC, num_subcores=NS,
    )
    b_per_w = n_chunks * CHUNK

    @functools.partial(
        pl.kernel,
        out_type=jax.ShapeDtypeStruct((NW * b_per_w, HDIM), jnp.float32),
        mesh=mesh,
        scratch_types=[
            pltpu.VMEM((n_chunks, CHUNK), jnp.int32),
            pltpu.VMEM((CHUNK, HDIM), jnp.float32),
            pltpu.SemaphoreType.DMA,
        ],
        compiler_params=pltpu.CompilerParams(use_tc_tiling_on_sc=False),
    )
    def k(x_hbm, table_hbm, out_hbm, idx_v, rows_v, gsem):
        wid = lax.axis_index("s") * NC + lax.axis_index("c")
        base = wid * b_per_w
        pltpu.sync_copy(x_hbm.at[wid], idx_v)

        def body(j, carry):
            pltpu.async_copy(table_hbm.at[idx_v.at[j]], rows_v, gsem).wait()
            pltpu.sync_copy(rows_v, out_hbm.at[pl.ds(base + j * CHUNK, CHUNK)])
            return carry

        lax.fori_loop(0, n_chunks, body, 0)

    return k(x3, weight)


def kernel(x, weight):
    B = x.shape[0] * x.shape[1]
    n_chunks = B // (NW * CHUNK)
    x3 = x.reshape(NW, n_chunks, CHUNK)
    out = _gather(x3, weight, n_chunks)
    return out.reshape(x.shape[0], x.shape[1], HDIM)


# speedup vs baseline: 1.1123x; 1.1123x over previous
import functools

import jax
import jax.numpy as jnp
from jax import lax
from jax.experimental import pallas as pl
from jax.experimental.pallas import tpu as pltpu
from jax.experimental.pallas import tpu_sc as plsc

HDIM = 64
NC = 2
NS = 16
NW = NC * NS
CHUNK = 128
NBUF = 8
PDIST = 4


@functools.partial(jax.jit, static_argnames=("n_chunks",))
def _gather(x3, weight, n_chunks):
    mesh = plsc.VectorSubcoreMesh(
        core_axis_name="c", subcore_axis_name="s",
        num_cores=NC, num_subcores=NS,
    )
    b_per_w = n_chunks * CHUNK
    assert n_chunks % NBUF == 0

    @functools.partial(
        pl.kernel,
        out_type=jax.ShapeDtypeStruct((NW * b_per_w, HDIM), jnp.float32),
        mesh=mesh,
        scratch_types=[
            pltpu.VMEM((n_chunks, CHUNK), jnp.int32),
            [pltpu.VMEM((CHUNK, HDIM), jnp.float32) for _ in range(NBUF)],
            [pltpu.SemaphoreType.DMA for _ in range(NBUF)],
            [pltpu.SemaphoreType.DMA for _ in range(NBUF)],
        ],
        compiler_params=pltpu.CompilerParams(use_tc_tiling_on_sc=False),
    )
    def k(x_hbm, table_hbm, out_hbm, idx_v, bufs, gsems, osems):
        wid = lax.axis_index("s") * NC + lax.axis_index("c")
        base = wid * b_per_w
        pltpu.sync_copy(x_hbm.at[wid], idx_v)

        for b in range(PDIST):
            pltpu.async_copy(table_hbm.at[idx_v.at[b]], bufs[b], gsems[b])

        def group(g, carry):
            for b in range(NBUF):
                j = g * NBUF + b
                jp = j + PDIST
                bp = (b + PDIST) % NBUF

                @pl.when(jnp.logical_and(jp >= NBUF, jp < n_chunks))
                def _():
                    pltpu.make_async_copy(
                        bufs[bp],
                        out_hbm.at[pl.ds(base + (jp - NBUF) * CHUNK, CHUNK)],
                        osems[bp],
                    ).wait()

                @pl.when(jp < n_chunks)
                def _():
                    pltpu.async_copy(table_hbm.at[idx_v.at[jp]], bufs[bp],
                                     gsems[bp])

                pltpu.make_async_copy(table_hbm.at[idx_v.at[j]], bufs[b],
                                      gsems[b]).wait()
                pltpu.async_copy(bufs[b],
                                 out_hbm.at[pl.ds(base + j * CHUNK, CHUNK)],
                                 osems[b])
            return carry

        lax.fori_loop(0, n_chunks // NBUF, group, 0)

        for b in range(NBUF):
            j = n_chunks - NBUF + b
            pltpu.make_async_copy(
                bufs[b], out_hbm.at[pl.ds(base + j * CHUNK, CHUNK)], osems[b]
            ).wait()

    return k(x3, weight)


def kernel(x, weight):
    B = x.shape[0] * x.shape[1]
    n_chunks = B // (NW * CHUNK)
    x3 = x.reshape(NW, n_chunks, CHUNK)
    out = _gather(x3, weight, n_chunks)
    return out.reshape(x.shape[0], x.shape[1], HDIM)


# speedup vs baseline: 1.1155x; 1.0028x over previous
import functools

import jax
import jax.numpy as jnp
from jax import lax
from jax.experimental import pallas as pl
from jax.experimental.pallas import tpu as pltpu
from jax.experimental.pallas import tpu_sc as plsc

HDIM = 64
NC = 2
NS = 16
NW = NC * NS
HALF_A = 104
HALF_B = 96
NBUF = 8
PDIST = 4


def _gather_kernel(B, S):
    rows_w = B // NW

    mesh = plsc.VectorSubcoreMesh(
        core_axis_name="c", subcore_axis_name="s",
        num_cores=NC, num_subcores=NS,
    )

    @functools.partial(
        pl.kernel,
        out_type=jax.ShapeDtypeStruct((B, S, HDIM), jnp.float32),
        mesh=mesh,
        scratch_types=[
            pltpu.VMEM((rows_w, S), jnp.int32),
            [pltpu.VMEM((S, HDIM), jnp.float32) for _ in range(NBUF)],
            [pltpu.SemaphoreType.DMA for _ in range(NBUF)],
            [pltpu.SemaphoreType.DMA for _ in range(NBUF)],
        ],
        compiler_params=pltpu.CompilerParams(use_tc_tiling_on_sc=False),
    )
    def k(x_hbm, table_hbm, out_hbm, idx_v, bufs, gsems, osems):
        wid = lax.axis_index("s") * NC + lax.axis_index("c")
        row0 = wid * rows_w
        pltpu.sync_copy(x_hbm.at[pl.ds(row0, rows_w)], idx_v)

        def fire(r, b):
            pltpu.async_copy(table_hbm.at[idx_v.at[r, pl.ds(0, HALF_A)]],
                             bufs[b].at[pl.ds(0, HALF_A)], gsems[b])
            pltpu.async_copy(table_hbm.at[idx_v.at[r, pl.ds(HALF_A, HALF_B)]],
                             bufs[b].at[pl.ds(HALF_A, HALF_B)], gsems[b])

        def wait_fire(b):
            pltpu.make_async_copy(table_hbm.at[idx_v.at[0, pl.ds(0, S)]],
                                  bufs[b], gsems[b]).wait()

        def wait_wb(r, b):
            pltpu.make_async_copy(bufs[b], out_hbm.at[row0 + r], osems[b]).wait()

        for b in range(PDIST):
            fire(b, b)

        def group(g, carry):
            for b in range(NBUF):
                r = g * NBUF + b
                rp = r + PDIST
                bp = (b + PDIST) % NBUF

                @pl.when(jnp.logical_and(rp >= NBUF, rp < rows_w))
                def _():
                    wait_wb(rp - NBUF, bp)

                @pl.when(rp < rows_w)
                def _():
                    fire(rp, bp)

                wait_fire(b)
                pltpu.async_copy(bufs[b], out_hbm.at[row0 + r], osems[b])
            return carry

        lax.fori_loop(0, rows_w // NBUF, group, 0)

        for b in range(NBUF):
            wait_wb(rows_w - NBUF + b, b)

    return k


def kernel(x, weight):
    B, S = x.shape
    return _gather_kernel(B, S)(x, weight)


# speedup vs baseline: 1.4854x; 1.3316x over previous
import functools

import jax
import jax.numpy as jnp
from jax import lax
from jax.experimental import pallas as pl
from jax.experimental.pallas import tpu as pltpu
from jax.experimental.pallas import tpu_sc as plsc

HDIM = 64
NC = 2
NS = 16
NW = NC * NS
HALF_A = 104
HALF_B = 96
NBUF = 8
PDIST = 4


def _gather_kernel(B, S):
    rows_w = B // NW

    mesh = plsc.VectorSubcoreMesh(
        core_axis_name="c", subcore_axis_name="s",
        num_cores=NC, num_subcores=NS,
    )

    @functools.partial(
        pl.kernel,
        out_type=jax.ShapeDtypeStruct((B * S, 2 * HDIM), jnp.float32),
        mesh=mesh,
        scratch_types=[
            pltpu.VMEM((rows_w, S), jnp.int32),
            [pltpu.VMEM((S, HDIM), jnp.float32) for _ in range(NBUF)],
            [pltpu.SemaphoreType.DMA for _ in range(NBUF)],
            [pltpu.SemaphoreType.DMA for _ in range(NBUF)],
        ],
        compiler_params=pltpu.CompilerParams(use_tc_tiling_on_sc=False),
    )
    def k(x_hbm, table_hbm, out_hbm, idx_v, bufs, gsems, osems):
        wid = lax.axis_index("s") * NC + lax.axis_index("c")
        row0 = wid * rows_w
        pltpu.sync_copy(x_hbm.at[pl.ds(row0, rows_w)], idx_v)

        def wb_dst(r):
            return out_hbm.at[pl.ds((row0 + r) * S, S), pl.ds(0, HDIM)]

        def fire(r, b):
            pltpu.async_copy(table_hbm.at[idx_v.at[r, pl.ds(0, HALF_A)]],
                             bufs[b].at[pl.ds(0, HALF_A)], gsems[b])
            pltpu.async_copy(table_hbm.at[idx_v.at[r, pl.ds(HALF_A, HALF_B)]],
                             bufs[b].at[pl.ds(HALF_A, HALF_B)], gsems[b])

        def wait_fire(b):
            pltpu.make_async_copy(table_hbm.at[idx_v.at[0, pl.ds(0, S)]],
                                  bufs[b], gsems[b]).wait()

        def wait_wb(r, b):
            pltpu.make_async_copy(bufs[b], wb_dst(r), osems[b]).wait()

        for b in range(PDIST):
            fire(b, b)

        def group(g, carry):
            for b in range(NBUF):
                r = g * NBUF + b
                rp = r + PDIST
                bp = (b + PDIST) % NBUF

                @pl.when(jnp.logical_and(rp >= NBUF, rp < rows_w))
                def _():
                    wait_wb(rp - NBUF, bp)

                @pl.when(rp < rows_w)
                def _():
                    fire(rp, bp)

                wait_fire(b)
                pltpu.async_copy(bufs[b], wb_dst(r), osems[b])
            return carry

        lax.fori_loop(0, rows_w // NBUF, group, 0)

        for b in range(NBUF):
            wait_wb(rows_w - NBUF + b, b)

    return k


def kernel(x, weight):
    B, S = x.shape
    out128 = _gather_kernel(B, S)(x, weight)
    return out128[:, :HDIM].reshape(B, S, HDIM)
